# Initial kernel scaffold; baseline (speedup 1.0000x reference)
import jax
import jax.numpy as jnp
from jax.experimental import pallas as pl


def kernel(x, edge_index, batch, W0, b0, W1, b1, W2, b2, lin_W, lin_b):
    raise NotImplementedError("write your pallas kernel here")


# speedup vs baseline: 19.0032x; 19.0032x over previous
import functools

import jax
import jax.numpy as jnp
from jax import lax
from jax.experimental import pallas as pl
from jax.experimental.pallas import tpu as pltpu
from jax.experimental.pallas import tpu_sc as plsc

N = 10000
E = 320000
F = 128
H = 32
NG = 128
NCLS = 10

NC, NS = 2, 16
NW = NC * NS
CW = 128
CHUNKS = 80
E_PAD = NW * CHUNKS * CW
N_PAD = 10240
RSL = N_PAD // NS
RB = 1280
GRID = N_PAD // RB

@functools.cache
def _mesh():
    return plsc.VectorSubcoreMesh(
        core_axis_name="c", subcore_axis_name="s",
        num_cores=NC, num_subcores=NS)


_sc_out = jax.ShapeDtypeStruct((NC, N_PAD, H), jnp.float32)


def _sc_scatter(g, srcw, dstw, zrows):
    @functools.partial(
        pl.kernel,
        out_type=_sc_out,
        mesh=_mesh(),
        scratch_types=[
            pltpu.VMEM((CHUNKS, CW), jnp.int32),
            pltpu.VMEM((CHUNKS, CW), jnp.int32),
            pltpu.VMEM((CW, H), jnp.float32),
            pltpu.VMEM_SHARED((N_PAD, H), jnp.float32),
            pltpu.SemaphoreType.DMA,
        ],
        compiler_params=pltpu.CompilerParams(use_tc_tiling_on_sc=False),
    )
    def k(g_hbm, src_hbm, dst_hbm, z_hbm, out_hbm, src_v, dst_v, rows_v, acc, sem):
        c = lax.axis_index("c")
        s = lax.axis_index("s")
        w = c * NS + s
        pltpu.sync_copy(z_hbm, acc.at[pl.ds(s * RSL, RSL)])
        pltpu.sync_copy(src_hbm.at[w], src_v)
        pltpu.sync_copy(dst_hbm.at[w], dst_v)
        plsc.subcore_barrier()

        def body(j, carry):
            pltpu.async_copy(g_hbm.at[src_v.at[j]], rows_v, sem).wait()
            pltpu.sync_copy(rows_v, acc.at[dst_v.at[j]], add=True)
            return carry

        lax.fori_loop(0, CHUNKS, body, 0)
        plsc.subcore_barrier()
        pltpu.sync_copy(acc.at[pl.ds(s * RSL, RSL)],
                        out_hbm.at[c, pl.ds(s * RSL, RSL)])

    return k(g, srcw, dstw, zrows)


def _sc_deg(ones_rows, dstw, zrows):
    @functools.partial(
        pl.kernel,
        out_type=_sc_out,
        mesh=_mesh(),
        scratch_types=[
            pltpu.VMEM((CHUNKS, CW), jnp.int32),
            pltpu.VMEM((CW, H), jnp.float32),
            pltpu.VMEM_SHARED((N_PAD, H), jnp.float32),
        ],
        compiler_params=pltpu.CompilerParams(use_tc_tiling_on_sc=False),
    )
    def k(ones_hbm, dst_hbm, z_hbm, out_hbm, dst_v, rows_v, acc):
        c = lax.axis_index("c")
        s = lax.axis_index("s")
        w = c * NS + s
        pltpu.sync_copy(z_hbm, acc.at[pl.ds(s * RSL, RSL)])
        pltpu.sync_copy(dst_hbm.at[w], dst_v)
        pltpu.sync_copy(ones_hbm, rows_v)
        plsc.subcore_barrier()

        def body(j, carry):
            pltpu.sync_copy(rows_v, acc.at[dst_v.at[j]], add=True)
            return carry

        lax.fori_loop(0, CHUNKS, body, 0)
        plsc.subcore_barrier()
        pltpu.sync_copy(acc.at[pl.ds(s * RSL, RSL)],
                        out_hbm.at[c, pl.ds(s * RSL, RSL)])

    return k(ones_rows, dstw, zrows)


def _tc_layer1(degp, xp, W0):
    def body(degp_ref, x_ref, w_ref, dis_ref, g_ref):
        dis = lax.rsqrt(degp_ref[0] + degp_ref[1] + 1.0)
        dis_ref[...] = dis
        h = jnp.dot(x_ref[...], w_ref[...],
                    preferred_element_type=jnp.float32)
        g_ref[...] = dis * h

    return pl.pallas_call(
        body,
        grid=(GRID,),
        in_specs=[
            pl.BlockSpec((NC, RB, H), lambda i: (0, i, 0)),
            pl.BlockSpec((RB, F), lambda i: (i, 0)),
            pl.BlockSpec((F, H), lambda i: (0, 0)),
        ],
        out_specs=[
            pl.BlockSpec((RB, H), lambda i: (i, 0)),
            pl.BlockSpec((RB, H), lambda i: (i, 0)),
        ],
        out_shape=[
            jax.ShapeDtypeStruct((N_PAD, H), jnp.float32),
            jax.ShapeDtypeStruct((N_PAD, H), jnp.float32),
        ],
    )(degp, xp, W0)


def _tc_layer(sp, g, dis, b, W):
    def body(s_ref, g_ref, dis_ref, b_ref, w_ref, out_ref):
        h = dis_ref[...] * (s_ref[0] + s_ref[1] + g_ref[...]) + b_ref[...]
        h = jnp.maximum(h, 0.0)
        out_ref[...] = dis_ref[...] * jnp.dot(
            h, w_ref[...], preferred_element_type=jnp.float32)

    return pl.pallas_call(
        body,
        grid=(GRID,),
        in_specs=[
            pl.BlockSpec((NC, RB, H), lambda i: (0, i, 0)),
            pl.BlockSpec((RB, H), lambda i: (i, 0)),
            pl.BlockSpec((RB, H), lambda i: (i, 0)),
            pl.BlockSpec((1, H), lambda i: (0, 0)),
            pl.BlockSpec((H, H), lambda i: (0, 0)),
        ],
        out_specs=pl.BlockSpec((RB, H), lambda i: (i, 0)),
        out_shape=jax.ShapeDtypeStruct((N_PAD, H), jnp.float32),
    )(sp, g, dis, b, W)


def _tc_head(sp, g, dis, b, batch_p, lin_W, lin_b):
    def body(s_ref, g_ref, dis_ref, b_ref, batch_ref, lw_ref, lb_ref,
             out_ref, acc_ref, cnt_ref):
        i = pl.program_id(0)

        @pl.when(i == 0)
        def _():
            acc_ref[...] = jnp.zeros_like(acc_ref)
            cnt_ref[...] = jnp.zeros_like(cnt_ref)

        h = dis_ref[...] * (s_ref[0] + s_ref[1] + g_ref[...]) + b_ref[...]
        h = jnp.maximum(h, 0.0)
        rows = lax.broadcasted_iota(jnp.int32, (RB, NG), 0) + i * RB
        gids = lax.broadcasted_iota(jnp.int32, (RB, NG), 1)
        onehot = jnp.where((batch_ref[...] == gids) & (rows < N), 1.0, 0.0)
        acc_ref[...] += lax.dot_general(
            onehot, h, (((0,), (0,)), ((), ())),
            preferred_element_type=jnp.float32)
        cnt_ref[...] += lax.dot_general(
            onehot, jnp.ones((RB, 1), jnp.float32), (((0,), (0,)), ((), ())),
            preferred_element_type=jnp.float32)

        @pl.when(i == GRID - 1)
        def _():
            pooled = acc_ref[...] / jnp.maximum(cnt_ref[...], 1.0)
            logits = jnp.dot(pooled, lw_ref[...],
                             preferred_element_type=jnp.float32) + lb_ref[...]
            logits = jnp.maximum(logits, 0.0)
            m = jnp.max(logits, axis=1, keepdims=True)
            lse = jnp.log(jnp.sum(jnp.exp(logits - m), axis=1,
                                  keepdims=True)) + m
            out_ref[...] = logits - lse

    return pl.pallas_call(
        body,
        grid=(GRID,),
        in_specs=[
            pl.BlockSpec((NC, RB, H), lambda i: (0, i, 0)),
            pl.BlockSpec((RB, H), lambda i: (i, 0)),
            pl.BlockSpec((RB, H), lambda i: (i, 0)),
            pl.BlockSpec((1, H), lambda i: (0, 0)),
            pl.BlockSpec((RB, 1), lambda i: (i, 0)),
            pl.BlockSpec((H, NCLS), lambda i: (0, 0)),
            pl.BlockSpec((1, NCLS), lambda i: (0, 0)),
        ],
        out_specs=pl.BlockSpec((NG, NCLS), lambda i: (0, 0)),
        out_shape=jax.ShapeDtypeStruct((NG, NCLS), jnp.float32),
        scratch_shapes=[
            pltpu.VMEM((NG, H), jnp.float32),
            pltpu.VMEM((NG, 1), jnp.float32),
        ],
    )(sp, g, dis, b, batch_p, lin_W, lin_b)


def kernel(x, edge_index, batch, W0, b0, W1, b1, W2, b2, lin_W, lin_b):
    src = edge_index[0].astype(jnp.int32)
    dst = edge_index[1].astype(jnp.int32)
    pad = jnp.full((E_PAD - E,), N, jnp.int32)
    srcw = jnp.concatenate([src, pad]).reshape(NW, CHUNKS, CW)
    dstw = jnp.concatenate([dst, pad]).reshape(NW, CHUNKS, CW)
    xp = jnp.pad(x, ((0, N_PAD - N), (0, 0)))
    batch_p = jnp.pad(batch.astype(jnp.int32), (0, N_PAD - N)).reshape(
        N_PAD, 1)
    zrows = jnp.zeros((RSL, H), jnp.float32)
    ones_rows = jnp.ones((CW, H), jnp.float32)

    degp = _sc_deg(ones_rows, dstw, zrows)
    dis, g1 = _tc_layer1(degp, xp, W0)
    s1 = _sc_scatter(g1, srcw, dstw, zrows)
    g2 = _tc_layer(s1, g1, dis, b0.reshape(1, H), W1)
    s2 = _sc_scatter(g2, srcw, dstw, zrows)
    g3 = _tc_layer(s2, g2, dis, b1.reshape(1, H), W2)
    s3 = _sc_scatter(g3, srcw, dstw, zrows)
    return _tc_head(s3, g3, dis, b2.reshape(1, H), batch_p,
                    lin_W, lin_b.reshape(1, NCLS))


# speedup vs baseline: 24.1702x; 1.2719x over previous
import functools

import jax
import jax.numpy as jnp
from jax import lax
from jax.experimental import pallas as pl
from jax.experimental.pallas import tpu as pltpu
from jax.experimental.pallas import tpu_sc as plsc

N = 10000
E = 320000
F = 128
H = 32
NG = 128
NCLS = 10

NC, NS = 2, 16
NW = NC * NS
CW = 128
CHUNKS = 80
E_PAD = NW * CHUNKS * CW
N_PAD = 10240
RSL = N_PAD // NS
RB = 1280
GRID = N_PAD // RB

@functools.cache
def _mesh():
    return plsc.VectorSubcoreMesh(
        core_axis_name="c", subcore_axis_name="s",
        num_cores=NC, num_subcores=NS)


_sc_out = jax.ShapeDtypeStruct((NC, N_PAD, H), jnp.float32)


def _sc_scatter(g, srcw, dstw, zrows):
    @functools.partial(
        pl.kernel,
        out_type=_sc_out,
        mesh=_mesh(),
        scratch_types=[
            pltpu.VMEM((CHUNKS, CW), jnp.int32),
            pltpu.VMEM((CHUNKS, CW), jnp.int32),
            pltpu.VMEM((CW, H), jnp.float32),
            pltpu.VMEM((CW, H), jnp.float32),
            pltpu.VMEM_SHARED((N_PAD, H), jnp.float32),
            pltpu.SemaphoreType.DMA,
        ],
        compiler_params=pltpu.CompilerParams(use_tc_tiling_on_sc=False),
    )
    def k(g_hbm, src_hbm, dst_hbm, z_hbm, out_hbm, src_v, dst_v,
          rows0_v, rows1_v, acc, sem):
        c = lax.axis_index("c")
        s = lax.axis_index("s")
        w = c * NS + s
        pltpu.sync_copy(z_hbm, acc.at[pl.ds(s * RSL, RSL)])
        pltpu.sync_copy(src_hbm.at[w], src_v)
        pltpu.sync_copy(dst_hbm.at[w], dst_v)
        plsc.subcore_barrier()

        pltpu.async_copy(g_hbm.at[src_v.at[0]], rows0_v, sem)
        pltpu.async_copy(g_hbm.at[src_v.at[1]], rows1_v, sem)

        def body(t2, carry):
            j = 2 * t2
            pltpu.make_async_copy(g_hbm.at[src_v.at[j]], rows0_v, sem).wait()
            pltpu.sync_copy(rows0_v, acc.at[dst_v.at[j]], add=True)

            @pl.when(j + 2 < CHUNKS)
            def _():
                pltpu.async_copy(g_hbm.at[src_v.at[j + 2]], rows0_v, sem)

            pltpu.make_async_copy(g_hbm.at[src_v.at[j]], rows1_v, sem).wait()
            pltpu.sync_copy(rows1_v, acc.at[dst_v.at[j + 1]], add=True)

            @pl.when(j + 3 < CHUNKS)
            def _():
                pltpu.async_copy(g_hbm.at[src_v.at[j + 3]], rows1_v, sem)

            return carry

        lax.fori_loop(0, CHUNKS // 2, body, 0)
        plsc.subcore_barrier()
        pltpu.sync_copy(acc.at[pl.ds(s * RSL, RSL)],
                        out_hbm.at[c, pl.ds(s * RSL, RSL)])

    return k(g, srcw, dstw, zrows)


def _sc_deg(ones_rows, dstw, zrows):
    @functools.partial(
        pl.kernel,
        out_type=_sc_out,
        mesh=_mesh(),
        scratch_types=[
            pltpu.VMEM((CHUNKS, CW), jnp.int32),
            pltpu.VMEM((CW, H), jnp.float32),
            pltpu.VMEM_SHARED((N_PAD, H), jnp.float32),
        ],
        compiler_params=pltpu.CompilerParams(use_tc_tiling_on_sc=False),
    )
    def k(ones_hbm, dst_hbm, z_hbm, out_hbm, dst_v, rows_v, acc):
        c = lax.axis_index("c")
        s = lax.axis_index("s")
        w = c * NS + s
        pltpu.sync_copy(z_hbm, acc.at[pl.ds(s * RSL, RSL)])
        pltpu.sync_copy(dst_hbm.at[w], dst_v)
        pltpu.sync_copy(ones_hbm, rows_v)
        plsc.subcore_barrier()

        def body(j, carry):
            pltpu.sync_copy(rows_v, acc.at[dst_v.at[j]], add=True)
            return carry

        lax.fori_loop(0, CHUNKS, body, 0)
        plsc.subcore_barrier()
        pltpu.sync_copy(acc.at[pl.ds(s * RSL, RSL)],
                        out_hbm.at[c, pl.ds(s * RSL, RSL)])

    return k(ones_rows, dstw, zrows)


def _tc_layer1(degp, xp, W0):
    def body(degp_ref, x_ref, w_ref, dis_ref, g_ref):
        dis = lax.rsqrt(degp_ref[0] + degp_ref[1] + 1.0)
        dis_ref[...] = dis
        h = jnp.dot(x_ref[...], w_ref[...],
                    preferred_element_type=jnp.float32)
        g_ref[...] = dis * h

    return pl.pallas_call(
        body,
        grid=(GRID,),
        in_specs=[
            pl.BlockSpec((NC, RB, H), lambda i: (0, i, 0)),
            pl.BlockSpec((RB, F), lambda i: (i, 0)),
            pl.BlockSpec((F, H), lambda i: (0, 0)),
        ],
        out_specs=[
            pl.BlockSpec((RB, H), lambda i: (i, 0)),
            pl.BlockSpec((RB, H), lambda i: (i, 0)),
        ],
        out_shape=[
            jax.ShapeDtypeStruct((N_PAD, H), jnp.float32),
            jax.ShapeDtypeStruct((N_PAD, H), jnp.float32),
        ],
    )(degp, xp, W0)


def _tc_layer(sp, g, dis, b, W):
    def body(s_ref, g_ref, dis_ref, b_ref, w_ref, out_ref):
        h = dis_ref[...] * (s_ref[0] + s_ref[1] + g_ref[...]) + b_ref[...]
        h = jnp.maximum(h, 0.0)
        out_ref[...] = dis_ref[...] * jnp.dot(
            h, w_ref[...], preferred_element_type=jnp.float32)

    return pl.pallas_call(
        body,
        grid=(GRID,),
        in_specs=[
            pl.BlockSpec((NC, RB, H), lambda i: (0, i, 0)),
            pl.BlockSpec((RB, H), lambda i: (i, 0)),
            pl.BlockSpec((RB, H), lambda i: (i, 0)),
            pl.BlockSpec((1, H), lambda i: (0, 0)),
            pl.BlockSpec((H, H), lambda i: (0, 0)),
        ],
        out_specs=pl.BlockSpec((RB, H), lambda i: (i, 0)),
        out_shape=jax.ShapeDtypeStruct((N_PAD, H), jnp.float32),
    )(sp, g, dis, b, W)


def _tc_head(sp, g, dis, b, batch_p, lin_W, lin_b):
    def body(s_ref, g_ref, dis_ref, b_ref, batch_ref, lw_ref, lb_ref,
             out_ref, acc_ref, cnt_ref):
        i = pl.program_id(0)

        @pl.when(i == 0)
        def _():
            acc_ref[...] = jnp.zeros_like(acc_ref)
            cnt_ref[...] = jnp.zeros_like(cnt_ref)

        h = dis_ref[...] * (s_ref[0] + s_ref[1] + g_ref[...]) + b_ref[...]
        h = jnp.maximum(h, 0.0)
        rows = lax.broadcasted_iota(jnp.int32, (RB, NG), 0) + i * RB
        gids = lax.broadcasted_iota(jnp.int32, (RB, NG), 1)
        onehot = jnp.where((batch_ref[...] == gids) & (rows < N), 1.0, 0.0)
        acc_ref[...] += lax.dot_general(
            onehot, h, (((0,), (0,)), ((), ())),
            preferred_element_type=jnp.float32)
        cnt_ref[...] += lax.dot_general(
            onehot, jnp.ones((RB, 1), jnp.float32), (((0,), (0,)), ((), ())),
            preferred_element_type=jnp.float32)

        @pl.when(i == GRID - 1)
        def _():
            pooled = acc_ref[...] / jnp.maximum(cnt_ref[...], 1.0)
            logits = jnp.dot(pooled, lw_ref[...],
                             preferred_element_type=jnp.float32) + lb_ref[...]
            logits = jnp.maximum(logits, 0.0)
            m = jnp.max(logits, axis=1, keepdims=True)
            lse = jnp.log(jnp.sum(jnp.exp(logits - m), axis=1,
                                  keepdims=True)) + m
            out_ref[...] = logits - lse

    return pl.pallas_call(
        body,
        grid=(GRID,),
        in_specs=[
            pl.BlockSpec((NC, RB, H), lambda i: (0, i, 0)),
            pl.BlockSpec((RB, H), lambda i: (i, 0)),
            pl.BlockSpec((RB, H), lambda i: (i, 0)),
            pl.BlockSpec((1, H), lambda i: (0, 0)),
            pl.BlockSpec((RB, 1), lambda i: (i, 0)),
            pl.BlockSpec((H, NCLS), lambda i: (0, 0)),
            pl.BlockSpec((1, NCLS), lambda i: (0, 0)),
        ],
        out_specs=pl.BlockSpec((NG, NCLS), lambda i: (0, 0)),
        out_shape=jax.ShapeDtypeStruct((NG, NCLS), jnp.float32),
        scratch_shapes=[
            pltpu.VMEM((NG, H), jnp.float32),
            pltpu.VMEM((NG, 1), jnp.float32),
        ],
    )(sp, g, dis, b, batch_p, lin_W, lin_b)


def kernel(x, edge_index, batch, W0, b0, W1, b1, W2, b2, lin_W, lin_b):
    src = edge_index[0].astype(jnp.int32)
    dst = edge_index[1].astype(jnp.int32)
    pad = jnp.full((E_PAD - E,), N, jnp.int32)
    srcw = jnp.concatenate([src, pad]).reshape(NW, CHUNKS, CW)
    dstw = jnp.concatenate([dst, pad]).reshape(NW, CHUNKS, CW)
    xp = jnp.pad(x, ((0, N_PAD - N), (0, 0)))
    batch_p = jnp.pad(batch.astype(jnp.int32), (0, N_PAD - N)).reshape(
        N_PAD, 1)
    zrows = jnp.zeros((RSL, H), jnp.float32)
    ones_rows = jnp.ones((CW, H), jnp.float32)

    degp = _sc_deg(ones_rows, dstw, zrows)
    dis, g1 = _tc_layer1(degp, xp, W0)
    s1 = _sc_scatter(g1, srcw, dstw, zrows)
    g2 = _tc_layer(s1, g1, dis, b0.reshape(1, H), W1)
    s2 = _sc_scatter(g2, srcw, dstw, zrows)
    g3 = _tc_layer(s2, g2, dis, b1.reshape(1, H), W2)
    s3 = _sc_scatter(g3, srcw, dstw, zrows)
    return _tc_head(s3, g3, dis, b2.reshape(1, H), batch_p,
                    lin_W, lin_b.reshape(1, NCLS))


# speedup vs baseline: 24.2489x; 1.0033x over previous
import functools

import jax
import jax.numpy as jnp
from jax import lax
from jax.experimental import pallas as pl
from jax.experimental.pallas import tpu as pltpu
from jax.experimental.pallas import tpu_sc as plsc

N = 10000
E = 320000
F = 128
H = 32
NG = 128
NCLS = 10

NC, NS = 2, 16
NW = NC * NS
CW = 128
CHUNKS = 80
E_PAD = NW * CHUNKS * CW
N_PAD = 10240
RSL = N_PAD // NS
RB = 1280
GRID = N_PAD // RB

@functools.cache
def _mesh():
    return plsc.VectorSubcoreMesh(
        core_axis_name="c", subcore_axis_name="s",
        num_cores=NC, num_subcores=NS)


_sc_out = jax.ShapeDtypeStruct((NC, N_PAD, H), jnp.float32)


def _sc_scatter(g, srcw, dstw, zrows):
    @functools.partial(
        pl.kernel,
        out_type=_sc_out,
        mesh=_mesh(),
        scratch_types=[
            pltpu.VMEM((CHUNKS, CW), jnp.int32),
            pltpu.VMEM((CHUNKS, CW), jnp.int32),
            pltpu.VMEM((4, CW, H), jnp.float32),
            pltpu.VMEM_SHARED((N_PAD, H), jnp.float32),
            pltpu.SemaphoreType.DMA,
            pltpu.SemaphoreType.DMA,
        ],
        compiler_params=pltpu.CompilerParams(use_tc_tiling_on_sc=False),
    )
    def k(g_hbm, src_hbm, dst_hbm, z_hbm, out_hbm, src_v, dst_v,
          rows_v, acc, sem_g, sem_s):
        c = lax.axis_index("c")
        s = lax.axis_index("s")
        w = c * NS + s
        pltpu.sync_copy(z_hbm, acc.at[pl.ds(s * RSL, RSL)])
        pltpu.sync_copy(src_hbm.at[w], src_v)
        pltpu.sync_copy(dst_hbm.at[w], dst_v)
        plsc.subcore_barrier()

        pltpu.async_copy(g_hbm.at[src_v.at[0]], rows_v.at[0], sem_g)
        pltpu.async_copy(g_hbm.at[src_v.at[1]], rows_v.at[1], sem_g)

        def step(j, b):
            pltpu.make_async_copy(
                g_hbm.at[src_v.at[0]], rows_v.at[b], sem_g).wait()
            pltpu.async_copy(rows_v.at[b], acc.at[dst_v.at[j]], sem_s,
                             add=True)

            @pl.when(j >= 2)
            def _():
                pltpu.make_async_copy(
                    rows_v.at[b], acc.at[dst_v.at[0]], sem_s).wait()

            @pl.when(j + 2 < CHUNKS)
            def _():
                pltpu.async_copy(
                    g_hbm.at[src_v.at[j + 2]], rows_v.at[(b + 2) % 4], sem_g)

        def body(t4, carry):
            j = 4 * t4
            step(j, 0)
            step(j + 1, 1)
            step(j + 2, 2)
            step(j + 3, 3)
            return carry

        lax.fori_loop(0, CHUNKS // 4, body, 0)
        pltpu.make_async_copy(rows_v.at[0], acc.at[dst_v.at[0]], sem_s).wait()
        pltpu.make_async_copy(rows_v.at[1], acc.at[dst_v.at[0]], sem_s).wait()
        plsc.subcore_barrier()
        pltpu.sync_copy(acc.at[pl.ds(s * RSL, RSL)],
                        out_hbm.at[c, pl.ds(s * RSL, RSL)])

    return k(g, srcw, dstw, zrows)


def _sc_deg(ones_rows, dstw, zrows):
    @functools.partial(
        pl.kernel,
        out_type=_sc_out,
        mesh=_mesh(),
        scratch_types=[
            pltpu.VMEM((CHUNKS, CW), jnp.int32),
            pltpu.VMEM((CW, H), jnp.float32),
            pltpu.VMEM_SHARED((N_PAD, H), jnp.float32),
            pltpu.SemaphoreType.DMA,
        ],
        compiler_params=pltpu.CompilerParams(use_tc_tiling_on_sc=False),
    )
    def k(ones_hbm, dst_hbm, z_hbm, out_hbm, dst_v, rows_v, acc, sem):
        c = lax.axis_index("c")
        s = lax.axis_index("s")
        w = c * NS + s
        pltpu.sync_copy(z_hbm, acc.at[pl.ds(s * RSL, RSL)])
        pltpu.sync_copy(dst_hbm.at[w], dst_v)
        pltpu.sync_copy(ones_hbm, rows_v)
        plsc.subcore_barrier()

        def body(j, carry):
            @pl.when(j >= 8)
            def _():
                pltpu.make_async_copy(
                    rows_v, acc.at[dst_v.at[0]], sem).wait()

            pltpu.async_copy(rows_v, acc.at[dst_v.at[j]], sem, add=True)
            return carry

        lax.fori_loop(0, CHUNKS, body, 0)
        def drain(j, carry):
            pltpu.make_async_copy(rows_v, acc.at[dst_v.at[0]], sem).wait()
            return carry

        lax.fori_loop(0, 8, drain, 0)
        plsc.subcore_barrier()
        pltpu.sync_copy(acc.at[pl.ds(s * RSL, RSL)],
                        out_hbm.at[c, pl.ds(s * RSL, RSL)])

    return k(ones_rows, dstw, zrows)


def _tc_layer1(degp, xp, W0):
    def body(degp_ref, x_ref, w_ref, dis_ref, g_ref):
        dis = lax.rsqrt(degp_ref[0] + degp_ref[1] + 1.0)
        dis_ref[...] = dis
        h = jnp.dot(x_ref[...], w_ref[...],
                    preferred_element_type=jnp.float32)
        g_ref[...] = dis * h

    return pl.pallas_call(
        body,
        grid=(GRID,),
        in_specs=[
            pl.BlockSpec((NC, RB, H), lambda i: (0, i, 0)),
            pl.BlockSpec((RB, F), lambda i: (i, 0)),
            pl.BlockSpec((F, H), lambda i: (0, 0)),
        ],
        out_specs=[
            pl.BlockSpec((RB, H), lambda i: (i, 0)),
            pl.BlockSpec((RB, H), lambda i: (i, 0)),
        ],
        out_shape=[
            jax.ShapeDtypeStruct((N_PAD, H), jnp.float32),
            jax.ShapeDtypeStruct((N_PAD, H), jnp.float32),
        ],
    )(degp, xp, W0)


def _tc_layer(sp, g, dis, b, W):
    def body(s_ref, g_ref, dis_ref, b_ref, w_ref, out_ref):
        h = dis_ref[...] * (s_ref[0] + s_ref[1] + g_ref[...]) + b_ref[...]
        h = jnp.maximum(h, 0.0)
        out_ref[...] = dis_ref[...] * jnp.dot(
            h, w_ref[...], preferred_element_type=jnp.float32)

    return pl.pallas_call(
        body,
        grid=(GRID,),
        in_specs=[
            pl.BlockSpec((NC, RB, H), lambda i: (0, i, 0)),
            pl.BlockSpec((RB, H), lambda i: (i, 0)),
            pl.BlockSpec((RB, H), lambda i: (i, 0)),
            pl.BlockSpec((1, H), lambda i: (0, 0)),
            pl.BlockSpec((H, H), lambda i: (0, 0)),
        ],
        out_specs=pl.BlockSpec((RB, H), lambda i: (i, 0)),
        out_shape=jax.ShapeDtypeStruct((N_PAD, H), jnp.float32),
    )(sp, g, dis, b, W)


def _tc_head(sp, g, dis, b, batch_p, lin_W, lin_b):
    def body(s_ref, g_ref, dis_ref, b_ref, batch_ref, lw_ref, lb_ref,
             out_ref, acc_ref, cnt_ref):
        i = pl.program_id(0)

        @pl.when(i == 0)
        def _():
            acc_ref[...] = jnp.zeros_like(acc_ref)
            cnt_ref[...] = jnp.zeros_like(cnt_ref)

        h = dis_ref[...] * (s_ref[0] + s_ref[1] + g_ref[...]) + b_ref[...]
        h = jnp.maximum(h, 0.0)
        rows = lax.broadcasted_iota(jnp.int32, (RB, NG), 0) + i * RB
        gids = lax.broadcasted_iota(jnp.int32, (RB, NG), 1)
        onehot = jnp.where((batch_ref[...] == gids) & (rows < N), 1.0, 0.0)
        acc_ref[...] += lax.dot_general(
            onehot, h, (((0,), (0,)), ((), ())),
            preferred_element_type=jnp.float32)
        cnt_ref[...] += lax.dot_general(
            onehot, jnp.ones((RB, 1), jnp.float32), (((0,), (0,)), ((), ())),
            preferred_element_type=jnp.float32)

        @pl.when(i == GRID - 1)
        def _():
            pooled = acc_ref[...] / jnp.maximum(cnt_ref[...], 1.0)
            logits = jnp.dot(pooled, lw_ref[...],
                             preferred_element_type=jnp.float32) + lb_ref[...]
            logits = jnp.maximum(logits, 0.0)
            m = jnp.max(logits, axis=1, keepdims=True)
            lse = jnp.log(jnp.sum(jnp.exp(logits - m), axis=1,
                                  keepdims=True)) + m
            out_ref[...] = logits - lse

    return pl.pallas_call(
        body,
        grid=(GRID,),
        in_specs=[
            pl.BlockSpec((NC, RB, H), lambda i: (0, i, 0)),
            pl.BlockSpec((RB, H), lambda i: (i, 0)),
            pl.BlockSpec((RB, H), lambda i: (i, 0)),
            pl.BlockSpec((1, H), lambda i: (0, 0)),
            pl.BlockSpec((RB, 1), lambda i: (i, 0)),
            pl.BlockSpec((H, NCLS), lambda i: (0, 0)),
            pl.BlockSpec((1, NCLS), lambda i: (0, 0)),
        ],
        out_specs=pl.BlockSpec((NG, NCLS), lambda i: (0, 0)),
        out_shape=jax.ShapeDtypeStruct((NG, NCLS), jnp.float32),
        scratch_shapes=[
            pltpu.VMEM((NG, H), jnp.float32),
            pltpu.VMEM((NG, 1), jnp.float32),
        ],
    )(sp, g, dis, b, batch_p, lin_W, lin_b)


def kernel(x, edge_index, batch, W0, b0, W1, b1, W2, b2, lin_W, lin_b):
    src = edge_index[0].astype(jnp.int32)
    dst = edge_index[1].astype(jnp.int32)
    pad = jnp.full((E_PAD - E,), N, jnp.int32)
    srcw = jnp.concatenate([src, pad]).reshape(NW, CHUNKS, CW)
    dstw = jnp.concatenate([dst, pad]).reshape(NW, CHUNKS, CW)
    xp = jnp.pad(x, ((0, N_PAD - N), (0, 0)))
    batch_p = jnp.pad(batch.astype(jnp.int32), (0, N_PAD - N)).reshape(
        N_PAD, 1)
    zrows = jnp.zeros((RSL, H), jnp.float32)
    ones_rows = jnp.ones((CW, H), jnp.float32)

    degp = _sc_deg(ones_rows, dstw, zrows)
    dis, g1 = _tc_layer1(degp, xp, W0)
    s1 = _sc_scatter(g1, srcw, dstw, zrows)
    g2 = _tc_layer(s1, g1, dis, b0.reshape(1, H), W1)
    s2 = _sc_scatter(g2, srcw, dstw, zrows)
    g3 = _tc_layer(s2, g2, dis, b1.reshape(1, H), W2)
    s3 = _sc_scatter(g3, srcw, dstw, zrows)
    return _tc_head(s3, g3, dis, b2.reshape(1, H), batch_p,
                    lin_W, lin_b.reshape(1, NCLS))
